# baseline (speedup 1.0000x reference)
import functools

import jax
import jax.numpy as jnp
from jax import lax
from jax.experimental import pallas as pl
from jax.experimental.pallas import tpu as pltpu
from jax.experimental.pallas import tpu_sc as plsc

U = 25000
NI = 25000
D = 64
DH = 32
E = 800000
NUM_LAYERS = 2

NC = 2
NS = 16
CH = 128
NRING = 6
IB = 6
LAG = 3
NB = 66
CHUNKS_PER_TILE = IB * NB
EDGES_PER_TILE = CH * CHUNKS_PER_TILE
E_PAD = EDGES_PER_TILE * NS
NCHUNK = E_PAD // CH
NPAD = 25088
ROWS_PER_TILE = NPAD // NS
DUMMY = 25000
DEG_NRING = 4

_mesh = plsc.VectorSubcoreMesh(
    core_axis_name="c", subcore_axis_name="s", num_cores=NC, num_subcores=NS
)
_sc_params = pltpu.CompilerParams(use_tc_tiling_on_sc=False)


@functools.partial(
    pl.kernel,
    out_type=jax.ShapeDtypeStruct((NC, NPAD), jnp.float32),
    mesh=_mesh,
    compiler_params=_sc_params,
    scratch_types=[
        pltpu.VMEM_SHARED((NPAD,), jnp.float32),
        pltpu.VMEM((2, IB, CH), jnp.int32),
        pltpu.VMEM((CH,), jnp.float32),
        pltpu.SemaphoreType.DMA((DEG_NRING,)),
    ],
)
def _degree_kernel(sidx_h, zeros1_h, deg_out, deg_sh, sidx_v, ones_v, ssems):
    c = lax.axis_index("c")
    s = lax.axis_index("s")
    base = s * CHUNKS_PER_TILE

    @pl.when(s == 0)
    def _():
        pltpu.sync_copy(zeros1_h, deg_sh)

    for q in range(CH // 16):
        ones_v[pl.ds(q * 16, 16)] = jnp.ones((16,), jnp.float32)
    plsc.subcore_barrier()

    def fire(pb, k):
        pltpu.async_copy(ones_v, deg_sh.at[sidx_v.at[pb].at[k]],
                         ssems.at[k % DEG_NRING], add=True)

    def swait(k):
        pltpu.make_async_copy(zeros1_h.at[pl.ds(0, CH)], ones_v,
                              ssems.at[k % DEG_NRING]).wait()

    pltpu.sync_copy(sidx_h.at[c].at[pl.ds(base, IB)], sidx_v.at[0])
    for k in range(IB):
        if k >= DEG_NRING:
            swait(k)
        fire(0, k)

    @pl.loop(1, NB)
    def _(b):
        p = lax.rem(b, 2)
        g0 = base + b * IB
        pltpu.sync_copy(sidx_h.at[c].at[pl.ds(g0, IB)], sidx_v.at[p])
        for k in range(IB):
            swait(k)
            fire(p, k)

    for k in range(DEG_NRING):
        swait(k)

    plsc.subcore_barrier()

    @pl.when(s == 0)
    def _():
        pltpu.sync_copy(deg_sh, deg_out.at[c])


@functools.partial(
    pl.kernel,
    out_type=[jax.ShapeDtypeStruct((NC, NPAD, DH), jnp.float32),
              jax.ShapeDtypeStruct((NC, NPAD, DH), jnp.float32)],
    mesh=_mesh,
    compiler_params=_sc_params,
    scratch_types=[
        pltpu.VMEM_SHARED((NPAD, DH), jnp.float32),
        pltpu.VMEM_SHARED((NPAD, DH), jnp.float32),
        pltpu.VMEM((NRING, CH, DH), jnp.float32),
        pltpu.VMEM((2, IB, CH), jnp.int32),
        pltpu.VMEM((2, IB, CH), jnp.int32),
        pltpu.SemaphoreType.DMA((NRING,)),
        pltpu.SemaphoreType.DMA((NRING,)),
    ],
)
def _layer_kernel(taba_h, tabb_h, gidx_h, sidx_h, zeros2_h, ma_out, mb_out,
                  acc_sh, tab_sh, rows_v, gidx_v, sidx_v, gsems, ssems):
    c = lax.axis_index("c")
    s = lax.axis_index("s")
    base = s * CHUNKS_PER_TILE
    rbase = s * ROWS_PER_TILE

    def gfire(pb, k):
        pltpu.async_copy(tab_sh.at[gidx_v.at[pb].at[k]],
                         rows_v.at[k % NRING], gsems.at[k % NRING])

    def gwait(k):
        pltpu.make_async_copy(zeros2_h.at[pl.ds(0, CH)], rows_v.at[k % NRING],
                              gsems.at[k % NRING]).wait()

    def sfire(pb, k):
        pltpu.async_copy(rows_v.at[k % NRING], acc_sh.at[sidx_v.at[pb].at[k]],
                         ssems.at[k % NRING], add=True)

    def swait(k):
        pltpu.make_async_copy(zeros2_h.at[pl.ds(0, CH)], rows_v.at[k % NRING],
                              ssems.at[k % NRING]).wait()

    def sweep(tab_h, m_out):
        pltpu.sync_copy(zeros2_h.at[pl.ds(rbase, ROWS_PER_TILE)],
                        acc_sh.at[pl.ds(rbase, ROWS_PER_TILE)])
        pltpu.sync_copy(tab_h.at[c].at[pl.ds(rbase, ROWS_PER_TILE)],
                        tab_sh.at[pl.ds(rbase, ROWS_PER_TILE)])
        plsc.subcore_barrier()

        pltpu.sync_copy(gidx_h.at[c].at[pl.ds(base, IB)], gidx_v.at[0])
        pltpu.sync_copy(sidx_h.at[c].at[pl.ds(base, IB)], sidx_v.at[0])
        for k in range(IB):
            gfire(0, k)
            if k >= LAG:
                gwait(k - LAG)
                sfire(0, k - LAG)

        @pl.loop(1, NB)
        def _(b):
            p = lax.rem(b, 2)
            pm = 1 - p
            g0 = base + b * IB
            pltpu.sync_copy(gidx_h.at[c].at[pl.ds(g0, IB)], gidx_v.at[p])
            pltpu.sync_copy(sidx_h.at[c].at[pl.ds(g0, IB)], sidx_v.at[p])
            for k in range(IB):
                swait(k)
                gfire(p, k)
                kk = k - LAG
                if kk >= 0:
                    gwait(kk)
                    sfire(p, kk)
                else:
                    gwait(kk + IB)
                    sfire(pm, kk + IB)

        ptail = (NB - 1) % 2
        for k in range(IB - LAG, IB):
            gwait(k)
            sfire(ptail, k)
        for k in range(NRING):
            swait(k)

        plsc.subcore_barrier()
        pltpu.sync_copy(acc_sh.at[pl.ds(rbase, ROWS_PER_TILE)],
                        m_out.at[c].at[pl.ds(rbase, ROWS_PER_TILE)])

    sweep(taba_h, ma_out)
    sweep(tabb_h, mb_out)


RB = 512
NRB = NPAD // RB


def _prep_body(fa_ref, fb_ref, deg_ref, ta_ref, tb_ref, inv_ref):
    deg = deg_ref[0, 0, :]
    inv = 1.0 / (jnp.sqrt(deg) + 1e-8)
    inv_ref[0, 0, :] = inv
    ta_ref[0] = fa_ref[0] * inv[:, None]
    tb_ref[0] = fb_ref[0] * inv[:, None]


def _tc_prep(fa, fb, deg):
    return pl.pallas_call(
        _prep_body,
        grid=(2, NRB),
        in_specs=[
            pl.BlockSpec((1, RB, DH), lambda cc, b: (1 - cc, b, 0)),
            pl.BlockSpec((1, RB, DH), lambda cc, b: (1 - cc, b, 0)),
            pl.BlockSpec((1, 1, RB), lambda cc, b: (1 - cc, 0, b)),
        ],
        out_specs=[
            pl.BlockSpec((1, RB, DH), lambda cc, b: (cc, b, 0)),
            pl.BlockSpec((1, RB, DH), lambda cc, b: (cc, b, 0)),
            pl.BlockSpec((1, 1, RB), lambda cc, b: (1 - cc, 0, b)),
        ],
        out_shape=[
            jax.ShapeDtypeStruct((NC, NPAD, DH), jnp.float32),
            jax.ShapeDtypeStruct((NC, NPAD, DH), jnp.float32),
            jax.ShapeDtypeStruct((NC, 1, NPAD), jnp.float32),
        ],
    )(fa, fb, deg)


def _mid_body(ma_ref, mb_ref, inv_ref, ta_ref, tb_ref):
    inv = inv_ref[0, 0, :]
    s2 = (inv * inv)[:, None]
    ta_ref[0] = ma_ref[0] * s2
    tb_ref[0] = mb_ref[0] * s2


def _tc_mid(ma, mb, inv):
    return pl.pallas_call(
        _mid_body,
        grid=(2, NRB),
        in_specs=[
            pl.BlockSpec((1, RB, DH), lambda cc, b: (1 - cc, b, 0)),
            pl.BlockSpec((1, RB, DH), lambda cc, b: (1 - cc, b, 0)),
            pl.BlockSpec((1, 1, RB), lambda cc, b: (1 - cc, 0, b)),
        ],
        out_specs=[
            pl.BlockSpec((1, RB, DH), lambda cc, b: (cc, b, 0)),
            pl.BlockSpec((1, RB, DH), lambda cc, b: (cc, b, 0)),
        ],
        out_shape=[
            jax.ShapeDtypeStruct((NC, NPAD, DH), jnp.float32),
            jax.ShapeDtypeStruct((NC, NPAD, DH), jnp.float32),
        ],
    )(ma, mb, inv)


def _final_body(fa_ref, fb_ref, ma1_ref, mb1_ref, ma2_ref, mb2_ref, inv_ref,
                oa_ref, ob_ref):
    inv = inv_ref[0, 0, :][:, None]
    scale = 1.0 / (NUM_LAYERS + 1)
    oa_ref[0] = (fa_ref[0] + (ma1_ref[0] + ma2_ref[0]) * inv) * scale
    ob_ref[0] = (fb_ref[0] + (mb1_ref[0] + mb2_ref[0]) * inv) * scale


def _tc_final(fa, fb, ma1, mb1, ma2, mb2, inv):
    bspec = pl.BlockSpec((1, RB, DH), lambda cc, b: (cc, b, 0))
    return pl.pallas_call(
        _final_body,
        grid=(2, NRB),
        in_specs=[bspec, bspec, bspec, bspec, bspec, bspec,
                  pl.BlockSpec((1, 1, RB), lambda cc, b: (cc, 0, b))],
        out_specs=[bspec, bspec],
        out_shape=[
            jax.ShapeDtypeStruct((NC, NPAD, DH), jnp.float32),
            jax.ShapeDtypeStruct((NC, NPAD, DH), jnp.float32),
        ],
    )(fa, fb, ma1, mb1, ma2, mb2, inv)


def kernel(users_feat, exercises_feat, edge_index):
    src = edge_index[0].astype(jnp.int32)
    dst = edge_index[1].astype(jnp.int32)
    pad = jnp.full((E_PAD - E,), DUMMY, dtype=jnp.int32)
    src_p = jnp.concatenate([src, pad])
    dst_p = jnp.concatenate([dst, pad])
    gidx = jnp.stack([dst_p, src_p]).reshape(NC, NCHUNK, CH)
    sidx = jnp.stack([src_p, dst_p]).reshape(NC, NCHUNK, CH)

    zrows = jnp.zeros((NPAD - U, DH), dtype=jnp.float32)
    fa = jnp.stack([jnp.concatenate([users_feat[:, :DH], zrows]),
                    jnp.concatenate([exercises_feat[:, :DH], zrows])])
    fb = jnp.stack([jnp.concatenate([users_feat[:, DH:], zrows]),
                    jnp.concatenate([exercises_feat[:, DH:], zrows])])
    zeros1 = jnp.zeros((NPAD,), dtype=jnp.float32)
    zeros2 = jnp.zeros((NPAD, DH), dtype=jnp.float32)

    deg = _degree_kernel(sidx, zeros1)
    deg3 = deg.reshape(NC, 1, NPAD)
    ta1, tb1, inv = _tc_prep(fa, fb, deg3)
    m1a, m1b = _layer_kernel(ta1, tb1, gidx, sidx, zeros2)
    ta2, tb2 = _tc_mid(m1a, m1b, inv)
    m2a, m2b = _layer_kernel(ta2, tb2, gidx, sidx, zeros2)
    oa, ob = _tc_final(fa, fb, m1a, m1b, m2a, m2b, inv)

    top = jnp.concatenate([oa[0, :U], ob[0, :U]], axis=1)
    bot = jnp.concatenate([oa[1, :NI], ob[1, :NI]], axis=1)
    return jnp.concatenate([top, bot], axis=0)


# speedup vs baseline: 1.0480x; 1.0480x over previous
import functools

import jax
import jax.numpy as jnp
from jax import lax
from jax.experimental import pallas as pl
from jax.experimental.pallas import tpu as pltpu
from jax.experimental.pallas import tpu_sc as plsc

U = 25000
NI = 25000
D = 64
DH = 32
E = 800000
NUM_LAYERS = 2

NC = 2
NS = 16
CH = 128
NRING = 6
IB = 6
LAG = 3
NB = 66
CHUNKS_PER_TILE = IB * NB
EDGES_PER_TILE = CH * CHUNKS_PER_TILE
E_PAD = EDGES_PER_TILE * NS
NCHUNK = E_PAD // CH
NPAD = 25088
ROWS_PER_TILE = NPAD // NS
DUMMY = 25000
DEG_NRING = 4

_mesh = plsc.VectorSubcoreMesh(
    core_axis_name="c", subcore_axis_name="s", num_cores=NC, num_subcores=NS
)
_sc_params = pltpu.CompilerParams(use_tc_tiling_on_sc=False)


@functools.partial(
    pl.kernel,
    out_type=jax.ShapeDtypeStruct((NC, NPAD), jnp.float32),
    mesh=_mesh,
    compiler_params=_sc_params,
    scratch_types=[
        pltpu.VMEM_SHARED((NPAD,), jnp.float32),
        pltpu.VMEM((2, IB, CH), jnp.int32),
        pltpu.VMEM((CH,), jnp.float32),
        pltpu.VMEM((ROWS_PER_TILE,), jnp.float32),
        pltpu.SemaphoreType.DMA((DEG_NRING,)),
    ],
)
def _degree_kernel(sidx_h, zeros1_h, deg_out, deg_sh, sidx_v, ones_v, zv, ssems):
    c = lax.axis_index("c")
    s = lax.axis_index("s")
    base = s * CHUNKS_PER_TILE

    @pl.loop(0, ROWS_PER_TILE // 16)
    def _(r):
        zv[pl.ds(r * 16, 16)] = jnp.zeros((16,), jnp.float32)
    pltpu.sync_copy(zv, deg_sh.at[pl.ds(s * ROWS_PER_TILE, ROWS_PER_TILE)])

    for q in range(CH // 16):
        ones_v[pl.ds(q * 16, 16)] = jnp.ones((16,), jnp.float32)
    plsc.subcore_barrier()

    def fire(pb, k):
        pltpu.async_copy(ones_v, deg_sh.at[sidx_v.at[pb].at[k]],
                         ssems.at[k % DEG_NRING], add=True)

    def swait(k):
        pltpu.make_async_copy(zeros1_h.at[pl.ds(0, CH)], ones_v,
                              ssems.at[k % DEG_NRING]).wait()

    pltpu.sync_copy(sidx_h.at[1 - c].at[pl.ds(base, IB)], sidx_v.at[0])
    for k in range(IB):
        if k >= DEG_NRING:
            swait(k)
        fire(0, k)

    @pl.loop(1, NB)
    def _(b):
        p = lax.rem(b, 2)
        g0 = base + b * IB
        pltpu.sync_copy(sidx_h.at[1 - c].at[pl.ds(g0, IB)], sidx_v.at[p])
        for k in range(IB):
            swait(k)
            fire(p, k)

    for k in range(DEG_NRING):
        swait(k)

    plsc.subcore_barrier()

    @pl.when(s == 0)
    def _():
        pltpu.sync_copy(deg_sh, deg_out.at[c])


@functools.partial(
    pl.kernel,
    out_type=jax.ShapeDtypeStruct((NC, NPAD, DH), jnp.float32),
    mesh=_mesh,
    compiler_params=_sc_params,
    scratch_types=[
        pltpu.VMEM_SHARED((NPAD, DH), jnp.float32),
        pltpu.VMEM_SHARED((NPAD, DH), jnp.float32),
        pltpu.VMEM((NRING, CH, DH), jnp.float32),
        pltpu.VMEM((2, IB, CH), jnp.int32),
        pltpu.VMEM((2, IB, CH), jnp.int32),
        pltpu.SemaphoreType.DMA((NRING,)),
        pltpu.SemaphoreType.DMA((NRING,)),
    ],
)
def _spmm_kernel(tab_h, edges_h, msg_out,
                 acc_sh, tab_sh, rows_v, gidx_v, sidx_v, gsems, ssems):
    c = lax.axis_index("c")
    s = lax.axis_index("s")
    base = s * CHUNKS_PER_TILE
    rbase = s * ROWS_PER_TILE

    @pl.loop(0, CH)
    def _(r):
        rows_v[0, r, pl.ds(0, 16)] = jnp.zeros((16,), jnp.float32)
        rows_v[0, r, pl.ds(16, 16)] = jnp.zeros((16,), jnp.float32)
    for j in range(ROWS_PER_TILE // CH):
        pltpu.sync_copy(rows_v.at[0], acc_sh.at[pl.ds(rbase + j * CH, CH)])
    pltpu.sync_copy(rows_v.at[0].at[pl.ds(0, ROWS_PER_TILE % CH)],
                    acc_sh.at[pl.ds(rbase + (ROWS_PER_TILE // CH) * CH,
                                    ROWS_PER_TILE % CH)])
    pltpu.sync_copy(tab_h.at[c].at[pl.ds(rbase, ROWS_PER_TILE)],
                    tab_sh.at[pl.ds(rbase, ROWS_PER_TILE)])
    plsc.subcore_barrier()

    def gfire(pb, k):
        pltpu.async_copy(tab_sh.at[gidx_v.at[pb].at[k]],
                         rows_v.at[k % NRING], gsems.at[k % NRING])

    def gwait(k):
        pltpu.make_async_copy(tab_h.at[c].at[pl.ds(0, CH)], rows_v.at[k % NRING],
                              gsems.at[k % NRING]).wait()

    def sfire(pb, k):
        pltpu.async_copy(rows_v.at[k % NRING], acc_sh.at[sidx_v.at[pb].at[k]],
                         ssems.at[k % NRING], add=True)

    def swait(k):
        pltpu.make_async_copy(tab_h.at[c].at[pl.ds(0, CH)], rows_v.at[k % NRING],
                              ssems.at[k % NRING]).wait()

    pltpu.sync_copy(edges_h.at[c].at[pl.ds(base, IB)], gidx_v.at[0])
    pltpu.sync_copy(edges_h.at[1 - c].at[pl.ds(base, IB)], sidx_v.at[0])
    for k in range(IB):
        gfire(0, k)
        if k >= LAG:
            gwait(k - LAG)
            sfire(0, k - LAG)

    @pl.loop(1, NB)
    def _(b):
        p = lax.rem(b, 2)
        pm = 1 - p
        g0 = base + b * IB
        pltpu.sync_copy(edges_h.at[c].at[pl.ds(g0, IB)], gidx_v.at[p])
        pltpu.sync_copy(edges_h.at[1 - c].at[pl.ds(g0, IB)], sidx_v.at[p])
        for k in range(IB):
            swait(k)
            gfire(p, k)
            kk = k - LAG
            if kk >= 0:
                gwait(kk)
                sfire(p, kk)
            else:
                gwait(kk + IB)
                sfire(pm, kk + IB)

    ptail = (NB - 1) % 2
    for k in range(IB - LAG, IB):
        gwait(k)
        sfire(ptail, k)
    for k in range(NRING):
        swait(k)

    plsc.subcore_barrier()
    pltpu.sync_copy(acc_sh.at[pl.ds(rbase, ROWS_PER_TILE)],
                    msg_out.at[c].at[pl.ds(rbase, ROWS_PER_TILE)])


RB = 512
NRB = NPAD // RB


def _prep_body(fa_ref, fb_ref, deg_ref, ta_ref, tb_ref, inv_ref):
    deg = deg_ref[0, 0, :]
    inv = 1.0 / (jnp.sqrt(deg) + 1e-8)
    inv_ref[0, 0, :] = inv
    ta_ref[0] = fa_ref[0] * inv[:, None]
    tb_ref[0] = fb_ref[0] * inv[:, None]


def _tc_prep(fa, fb, deg):
    return pl.pallas_call(
        _prep_body,
        grid=(2, NRB),
        in_specs=[
            pl.BlockSpec((1, RB, DH), lambda cc, b: (1 - cc, b, 0)),
            pl.BlockSpec((1, RB, DH), lambda cc, b: (1 - cc, b, 0)),
            pl.BlockSpec((1, 1, RB), lambda cc, b: (1 - cc, 0, b)),
        ],
        out_specs=[
            pl.BlockSpec((1, RB, DH), lambda cc, b: (cc, b, 0)),
            pl.BlockSpec((1, RB, DH), lambda cc, b: (cc, b, 0)),
            pl.BlockSpec((1, 1, RB), lambda cc, b: (1 - cc, 0, b)),
        ],
        out_shape=[
            jax.ShapeDtypeStruct((NC, NPAD, DH), jnp.float32),
            jax.ShapeDtypeStruct((NC, NPAD, DH), jnp.float32),
            jax.ShapeDtypeStruct((NC, 1, NPAD), jnp.float32),
        ],
    )(fa, fb, deg)


def _mid_body(ma_ref, mb_ref, inv_ref, ta_ref, tb_ref):
    inv = inv_ref[0, 0, :]
    s2 = (inv * inv)[:, None]
    ta_ref[0] = ma_ref[0] * s2
    tb_ref[0] = mb_ref[0] * s2


def _tc_mid(ma, mb, inv):
    return pl.pallas_call(
        _mid_body,
        grid=(2, NRB),
        in_specs=[
            pl.BlockSpec((1, RB, DH), lambda cc, b: (1 - cc, b, 0)),
            pl.BlockSpec((1, RB, DH), lambda cc, b: (1 - cc, b, 0)),
            pl.BlockSpec((1, 1, RB), lambda cc, b: (1 - cc, 0, b)),
        ],
        out_specs=[
            pl.BlockSpec((1, RB, DH), lambda cc, b: (cc, b, 0)),
            pl.BlockSpec((1, RB, DH), lambda cc, b: (cc, b, 0)),
        ],
        out_shape=[
            jax.ShapeDtypeStruct((NC, NPAD, DH), jnp.float32),
            jax.ShapeDtypeStruct((NC, NPAD, DH), jnp.float32),
        ],
    )(ma, mb, inv)


def _final_body(fa_ref, fb_ref, ma1_ref, mb1_ref, ma2_ref, mb2_ref, inv_ref,
                oa_ref, ob_ref):
    inv = inv_ref[0, 0, :][:, None]
    scale = 1.0 / (NUM_LAYERS + 1)
    oa_ref[0] = (fa_ref[0] + (ma1_ref[0] + ma2_ref[0]) * inv) * scale
    ob_ref[0] = (fb_ref[0] + (mb1_ref[0] + mb2_ref[0]) * inv) * scale


def _tc_final(fa, fb, ma1, mb1, ma2, mb2, inv):
    bspec = pl.BlockSpec((1, RB, DH), lambda cc, b: (cc, b, 0))
    return pl.pallas_call(
        _final_body,
        grid=(2, NRB),
        in_specs=[bspec, bspec, bspec, bspec, bspec, bspec,
                  pl.BlockSpec((1, 1, RB), lambda cc, b: (cc, 0, b))],
        out_specs=[bspec, bspec],
        out_shape=[
            jax.ShapeDtypeStruct((NC, NPAD, DH), jnp.float32),
            jax.ShapeDtypeStruct((NC, NPAD, DH), jnp.float32),
        ],
    )(fa, fb, ma1, mb1, ma2, mb2, inv)


def kernel(users_feat, exercises_feat, edge_index):
    src = edge_index[0].astype(jnp.int32)
    dst = edge_index[1].astype(jnp.int32)
    pad = jnp.full((E_PAD - E,), DUMMY, dtype=jnp.int32)
    src_p = jnp.concatenate([src, pad])
    dst_p = jnp.concatenate([dst, pad])
    edges2 = jnp.stack([dst_p, src_p]).reshape(NC, NCHUNK, CH)

    zrows = jnp.zeros((NPAD - U, DH), dtype=jnp.float32)
    fa = jnp.stack([jnp.concatenate([users_feat[:, :DH], zrows]),
                    jnp.concatenate([exercises_feat[:, :DH], zrows])])
    fb = jnp.stack([jnp.concatenate([users_feat[:, DH:], zrows]),
                    jnp.concatenate([exercises_feat[:, DH:], zrows])])
    zeros1 = jnp.zeros((NPAD,), dtype=jnp.float32)

    deg = _degree_kernel(edges2, zeros1)
    deg3 = deg.reshape(NC, 1, NPAD)
    ta1, tb1, inv = _tc_prep(fa, fb, deg3)
    m1a = _spmm_kernel(ta1, edges2)
    m1b = _spmm_kernel(tb1, edges2)
    ta2, tb2 = _tc_mid(m1a, m1b, inv)
    m2a = _spmm_kernel(ta2, edges2)
    m2b = _spmm_kernel(tb2, edges2)
    oa, ob = _tc_final(fa, fb, m1a, m1b, m2a, m2b, inv)

    top = jnp.concatenate([oa[0, :U], ob[0, :U]], axis=1)
    bot = jnp.concatenate([oa[1, :NI], ob[1, :NI]], axis=1)
    return jnp.concatenate([top, bot], axis=0)


# speedup vs baseline: 1.1136x; 1.0626x over previous
import functools

import jax
import jax.numpy as jnp
from jax import lax
from jax.experimental import pallas as pl
from jax.experimental.pallas import tpu as pltpu
from jax.experimental.pallas import tpu_sc as plsc

U = 25000
NI = 25000
D = 64
DH = 32
E = 800000
NUM_LAYERS = 2

NC = 2
NS = 16
CH = 128
NRING = 6
IB = 6
LAG = 3
NB = 66
CHUNKS_PER_TILE = IB * NB
EDGES_PER_TILE = CH * CHUNKS_PER_TILE
E_PAD = EDGES_PER_TILE * NS
NCHUNK = E_PAD // CH
NPAD = 25088
ROWS_PER_TILE = NPAD // NS
DUMMY = 25000
DEG_NRING = 4

_mesh = plsc.VectorSubcoreMesh(
    core_axis_name="c", subcore_axis_name="s", num_cores=NC, num_subcores=NS
)
_sc_params = pltpu.CompilerParams(use_tc_tiling_on_sc=False)


@functools.partial(
    pl.kernel,
    out_type=jax.ShapeDtypeStruct((NC, NPAD), jnp.float32),
    mesh=_mesh,
    compiler_params=_sc_params,
    scratch_types=[
        pltpu.VMEM_SHARED((NPAD,), jnp.float32),
        pltpu.VMEM((2, IB, CH), jnp.int32),
        pltpu.VMEM((CH,), jnp.float32),
        pltpu.VMEM((ROWS_PER_TILE,), jnp.float32),
        pltpu.SemaphoreType.DMA((DEG_NRING,)),
    ],
)
def _degree_kernel(sidx_h, zeros1_h, deg_out, deg_sh, sidx_v, ones_v, zv, ssems):
    c = lax.axis_index("c")
    s = lax.axis_index("s")
    base = s * CHUNKS_PER_TILE

    @pl.loop(0, ROWS_PER_TILE // 16)
    def _(r):
        zv[pl.ds(r * 16, 16)] = jnp.zeros((16,), jnp.float32)
    pltpu.sync_copy(zv, deg_sh.at[pl.ds(s * ROWS_PER_TILE, ROWS_PER_TILE)])

    for q in range(CH // 16):
        ones_v[pl.ds(q * 16, 16)] = jnp.ones((16,), jnp.float32)
    plsc.subcore_barrier()

    def fire(pb, k):
        pltpu.async_copy(ones_v, deg_sh.at[sidx_v.at[pb].at[k]],
                         ssems.at[k % DEG_NRING], add=True)

    def swait(k):
        pltpu.make_async_copy(zeros1_h.at[pl.ds(0, CH)], ones_v,
                              ssems.at[k % DEG_NRING]).wait()

    pltpu.sync_copy(sidx_h.at[1 - c].at[pl.ds(base, IB)], sidx_v.at[0])
    for k in range(IB):
        if k >= DEG_NRING:
            swait(k)
        fire(0, k)

    @pl.loop(1, NB)
    def _(b):
        p = lax.rem(b, 2)
        g0 = base + b * IB
        pltpu.sync_copy(sidx_h.at[1 - c].at[pl.ds(g0, IB)], sidx_v.at[p])
        for k in range(IB):
            swait(k)
            fire(p, k)

    for k in range(DEG_NRING):
        swait(k)

    plsc.subcore_barrier()

    @pl.when(s == 0)
    def _():
        pltpu.sync_copy(deg_sh, deg_out.at[c])


@functools.partial(
    pl.kernel,
    out_type=jax.ShapeDtypeStruct((NC, NPAD, DH), jnp.float32),
    mesh=_mesh,
    compiler_params=_sc_params,
    scratch_types=[
        pltpu.VMEM_SHARED((NPAD, DH), jnp.float32),
        pltpu.VMEM_SHARED((NPAD, DH), jnp.float32),
        pltpu.VMEM((NRING, CH, DH), jnp.float32),
        pltpu.VMEM((2, IB, CH), jnp.int32),
        pltpu.VMEM((2, IB, CH), jnp.int32),
        pltpu.SemaphoreType.DMA((NRING,)),
        pltpu.SemaphoreType.DMA((NRING,)),
        pltpu.SemaphoreType.DMA,
    ],
)
def _spmm_kernel(tab_h, edges_h, msg_out,
                 acc_sh, tab_sh, rows_v, gidx_v, sidx_v, gsems, ssems, isem):
    c = lax.axis_index("c")
    s = lax.axis_index("s")
    base = s * CHUNKS_PER_TILE
    rbase = s * ROWS_PER_TILE

    @pl.loop(0, CH)
    def _(r):
        rows_v[0, r, pl.ds(0, 16)] = jnp.zeros((16,), jnp.float32)
        rows_v[0, r, pl.ds(16, 16)] = jnp.zeros((16,), jnp.float32)
    for j in range(ROWS_PER_TILE // CH):
        pltpu.sync_copy(rows_v.at[0], acc_sh.at[pl.ds(rbase + j * CH, CH)])
    pltpu.sync_copy(rows_v.at[0].at[pl.ds(0, ROWS_PER_TILE % CH)],
                    acc_sh.at[pl.ds(rbase + (ROWS_PER_TILE // CH) * CH,
                                    ROWS_PER_TILE % CH)])
    pltpu.sync_copy(tab_h.at[c].at[pl.ds(rbase, ROWS_PER_TILE)],
                    tab_sh.at[pl.ds(rbase, ROWS_PER_TILE)])
    plsc.subcore_barrier()

    def gfire(pb, k):
        pltpu.async_copy(tab_sh.at[gidx_v.at[pb].at[k]],
                         rows_v.at[k % NRING], gsems.at[k % NRING])

    def gwait(k):
        pltpu.make_async_copy(tab_h.at[c].at[pl.ds(0, CH)], rows_v.at[k % NRING],
                              gsems.at[k % NRING]).wait()

    def sfire(pb, k):
        pltpu.async_copy(rows_v.at[k % NRING], acc_sh.at[sidx_v.at[pb].at[k]],
                         ssems.at[k % NRING], add=True)

    def swait(k):
        pltpu.make_async_copy(tab_h.at[c].at[pl.ds(0, CH)], rows_v.at[k % NRING],
                              ssems.at[k % NRING]).wait()

    def iwait():
        pltpu.make_async_copy(edges_h.at[0].at[pl.ds(0, IB)], gidx_v.at[0],
                              isem).wait()
        pltpu.make_async_copy(edges_h.at[0].at[pl.ds(0, IB)], sidx_v.at[0],
                              isem).wait()

    def iprefetch(pb, b):
        g0 = base + b * IB
        pltpu.async_copy(edges_h.at[c].at[pl.ds(g0, IB)], gidx_v.at[pb], isem)
        pltpu.async_copy(edges_h.at[1 - c].at[pl.ds(g0, IB)], sidx_v.at[pb],
                         isem)

    pltpu.sync_copy(edges_h.at[c].at[pl.ds(base, IB)], gidx_v.at[0])
    pltpu.sync_copy(edges_h.at[1 - c].at[pl.ds(base, IB)], sidx_v.at[0])
    iprefetch(1, 1)
    for k in range(IB):
        gfire(0, k)
        if k >= LAG:
            gwait(k - LAG)
            sfire(0, k - LAG)

    @pl.loop(1, NB)
    def _(b):
        p = lax.rem(b, 2)
        pm = 1 - p
        iwait()
        for k in range(IB):
            swait(k)
            gfire(p, k)
            kk = k - LAG
            if kk >= 0:
                gwait(kk)
                sfire(p, kk)
            else:
                gwait(kk + IB)
                sfire(pm, kk + IB)

        @pl.when(b < NB - 1)
        def _():
            iprefetch(pm, b + 1)

    ptail = (NB - 1) % 2
    for k in range(IB - LAG, IB):
        gwait(k)
        sfire(ptail, k)
    for k in range(NRING):
        swait(k)

    plsc.subcore_barrier()
    pltpu.sync_copy(acc_sh.at[pl.ds(rbase, ROWS_PER_TILE)],
                    msg_out.at[c].at[pl.ds(rbase, ROWS_PER_TILE)])


RB = 512
NRB = NPAD // RB


def _prep_body(fa_ref, fb_ref, deg_ref, ta_ref, tb_ref, inv_ref):
    deg = deg_ref[0, 0, :]
    inv = 1.0 / (jnp.sqrt(deg) + 1e-8)
    inv_ref[0, 0, :] = inv
    ta_ref[0] = fa_ref[0] * inv[:, None]
    tb_ref[0] = fb_ref[0] * inv[:, None]


def _tc_prep(fa, fb, deg):
    return pl.pallas_call(
        _prep_body,
        grid=(2, NRB),
        in_specs=[
            pl.BlockSpec((1, RB, DH), lambda cc, b: (1 - cc, b, 0)),
            pl.BlockSpec((1, RB, DH), lambda cc, b: (1 - cc, b, 0)),
            pl.BlockSpec((1, 1, RB), lambda cc, b: (1 - cc, 0, b)),
        ],
        out_specs=[
            pl.BlockSpec((1, RB, DH), lambda cc, b: (cc, b, 0)),
            pl.BlockSpec((1, RB, DH), lambda cc, b: (cc, b, 0)),
            pl.BlockSpec((1, 1, RB), lambda cc, b: (1 - cc, 0, b)),
        ],
        out_shape=[
            jax.ShapeDtypeStruct((NC, NPAD, DH), jnp.float32),
            jax.ShapeDtypeStruct((NC, NPAD, DH), jnp.float32),
            jax.ShapeDtypeStruct((NC, 1, NPAD), jnp.float32),
        ],
    )(fa, fb, deg)


def _mid_body(ma_ref, mb_ref, inv_ref, ta_ref, tb_ref):
    inv = inv_ref[0, 0, :]
    s2 = (inv * inv)[:, None]
    ta_ref[0] = ma_ref[0] * s2
    tb_ref[0] = mb_ref[0] * s2


def _tc_mid(ma, mb, inv):
    return pl.pallas_call(
        _mid_body,
        grid=(2, NRB),
        in_specs=[
            pl.BlockSpec((1, RB, DH), lambda cc, b: (1 - cc, b, 0)),
            pl.BlockSpec((1, RB, DH), lambda cc, b: (1 - cc, b, 0)),
            pl.BlockSpec((1, 1, RB), lambda cc, b: (1 - cc, 0, b)),
        ],
        out_specs=[
            pl.BlockSpec((1, RB, DH), lambda cc, b: (cc, b, 0)),
            pl.BlockSpec((1, RB, DH), lambda cc, b: (cc, b, 0)),
        ],
        out_shape=[
            jax.ShapeDtypeStruct((NC, NPAD, DH), jnp.float32),
            jax.ShapeDtypeStruct((NC, NPAD, DH), jnp.float32),
        ],
    )(ma, mb, inv)


def _final_body(fa_ref, fb_ref, ma1_ref, mb1_ref, ma2_ref, mb2_ref, inv_ref,
                oa_ref, ob_ref):
    inv = inv_ref[0, 0, :][:, None]
    scale = 1.0 / (NUM_LAYERS + 1)
    oa_ref[0] = (fa_ref[0] + (ma1_ref[0] + ma2_ref[0]) * inv) * scale
    ob_ref[0] = (fb_ref[0] + (mb1_ref[0] + mb2_ref[0]) * inv) * scale


def _tc_final(fa, fb, ma1, mb1, ma2, mb2, inv):
    bspec = pl.BlockSpec((1, RB, DH), lambda cc, b: (cc, b, 0))
    return pl.pallas_call(
        _final_body,
        grid=(2, NRB),
        in_specs=[bspec, bspec, bspec, bspec, bspec, bspec,
                  pl.BlockSpec((1, 1, RB), lambda cc, b: (cc, 0, b))],
        out_specs=[bspec, bspec],
        out_shape=[
            jax.ShapeDtypeStruct((NC, NPAD, DH), jnp.float32),
            jax.ShapeDtypeStruct((NC, NPAD, DH), jnp.float32),
        ],
    )(fa, fb, ma1, mb1, ma2, mb2, inv)


def kernel(users_feat, exercises_feat, edge_index):
    src = edge_index[0].astype(jnp.int32)
    dst = edge_index[1].astype(jnp.int32)
    pad = jnp.full((E_PAD - E,), DUMMY, dtype=jnp.int32)
    src_p = jnp.concatenate([src, pad])
    dst_p = jnp.concatenate([dst, pad])
    edges2 = jnp.stack([dst_p, src_p]).reshape(NC, NCHUNK, CH)

    zrows = jnp.zeros((NPAD - U, DH), dtype=jnp.float32)
    fa = jnp.stack([jnp.concatenate([users_feat[:, :DH], zrows]),
                    jnp.concatenate([exercises_feat[:, :DH], zrows])])
    fb = jnp.stack([jnp.concatenate([users_feat[:, DH:], zrows]),
                    jnp.concatenate([exercises_feat[:, DH:], zrows])])
    zeros1 = jnp.zeros((NPAD,), dtype=jnp.float32)

    deg = _degree_kernel(edges2, zeros1)
    deg3 = deg.reshape(NC, 1, NPAD)
    ta1, tb1, inv = _tc_prep(fa, fb, deg3)
    m1a = _spmm_kernel(ta1, edges2)
    m1b = _spmm_kernel(tb1, edges2)
    ta2, tb2 = _tc_mid(m1a, m1b, inv)
    m2a = _spmm_kernel(ta2, edges2)
    m2b = _spmm_kernel(tb2, edges2)
    oa, ob = _tc_final(fa, fb, m1a, m1b, m2a, m2b, inv)

    top = jnp.concatenate([oa[0, :U], ob[0, :U]], axis=1)
    bot = jnp.concatenate([oa[1, :NI], ob[1, :NI]], axis=1)
    return jnp.concatenate([top, bot], axis=0)
